# baseline (speedup 1.0000x reference)
import jax
import jax.numpy as jnp
from jax.experimental import pallas as pl
from jax.experimental.pallas import tpu as pltpu

_EPS = 1e-9
_DECAY = 0.99

_W = 256


_WPB = 4


def _fused_body(x_ref, sc_ref, mean_ref, var_ref, ent_ref):
    b = pl.program_id(0)
    s = pl.program_id(1)
    nblk = pl.num_programs(1)

    for w in range(_WPB):
        x = x_ref[0, w * _W:(w + 1) * _W, :]

        lacc = x[:, 0:128]
        for c in range(1, x.shape[1] // 128):
            lacc = lacc + x[:, c * 128:(c + 1) * 128]
        s8 = lacc[:, 0:8]
        for k in range(1, 16):
            s8 = s8 + lacc[:, k * 8:(k + 1) * 8]
        a4 = s8[:, 0:4] + s8[:, 4:8]
        a2 = a4[:, 0:2] + a4[:, 2:4]
        tp = (a2[:, 0:1] + a2[:, 1:2]) + _EPS
        attn = x / tp

        acc = attn[0:8, :]
        for k in range(1, _W // 8):
            acc = acc + attn[k * 8:(k + 1) * 8, :]
        a = acc[0:4, :] + acc[4:8, :]
        a = a[0:2, :] + a[2:4, :]
        part = a[0:1, :] + a[1:2, :]

        @pl.when((s == 0) & (w == 0))
        def _():
            sc_ref[0] = part

        if w == 0:
            @pl.when(s > 0)
            def _():
                sc_ref[0] = sc_ref[0] + part
        else:
            sc_ref[0] = sc_ref[0] + part

        e = -jnp.sum(attn * jnp.log(attn + _EPS), axis=-1)
        ent_ref[b * nblk + s, w * _W:(w + 1) * _W] = e

    @pl.when((b == pl.num_programs(0) - 1) & (s == nblk - 1))
    def _():
        ent = ent_ref[...]
        n = ent.size
        mu = jnp.sum(ent) / n
        d = ent - mu
        mean_ref[...] = jnp.reshape(mu, (1, 1))
        var_ref[...] = jnp.reshape(jnp.sum(d * d) / n, (1, 1))


def _fused_scores_entropy(aw):
    B, S, _ = aw.shape
    nblk = S // (_W * _WPB)
    scores, mean, var = pl.pallas_call(
        _fused_body,
        grid=(B, nblk),
        in_specs=[
            pl.BlockSpec((1, _W * _WPB, S), lambda b, s: (b, s, 0)),
        ],
        out_specs=[
            pl.BlockSpec((1, 1, S), lambda b, s: (b, 0, 0)),
            pl.BlockSpec((1, 1), lambda b, s: (0, 0)),
            pl.BlockSpec((1, 1), lambda b, s: (0, 0)),
        ],
        out_shape=[
            jax.ShapeDtypeStruct((B, 1, S), jnp.float32),
            jax.ShapeDtypeStruct((1, 1), jnp.float32),
            jax.ShapeDtypeStruct((1, 1), jnp.float32),
        ],
        scratch_shapes=[pltpu.VMEM((B * nblk, _W * _WPB), jnp.float32)],
    )(aw)
    return scores.reshape(B, S), mean[0, 0], var[0, 0]


def _tail_body(idx_smem, idxv_ref, hid_ref, mk_ref, mv_ref,
               sk_ref, ok_ref, ov_ref, sem):
    n_tok = sk_ref.shape[0]
    m = mk_ref.shape[0]

    def start(i, _):
        row = idx_smem[i]
        pltpu.make_async_copy(hid_ref.at[pl.ds(row, 1), :],
                              sk_ref.at[pl.ds(i, 1), :], sem).start()
        return 0
    jax.lax.fori_loop(0, n_tok, start, 0)

    def wait(i, _):
        pltpu.make_async_copy(hid_ref.at[pl.ds(0, 1), :],
                              sk_ref.at[pl.ds(0, 1), :], sem).wait()
        return 0
    jax.lax.fori_loop(0, n_tok, wait, 0)

    idx = idxv_ref[...]
    slots = jax.lax.rem(idx, m)
    rows = jax.lax.broadcasted_iota(jnp.int32, (m, n_tok), 0)
    oh = (rows == slots).astype(jnp.float32)
    toks = sk_ref[...]
    sums = jnp.dot(oh, toks, preferred_element_type=jnp.float32)
    counts = jnp.sum(oh, axis=1, keepdims=True)
    means = sums / jnp.maximum(counts, 1.0)
    written = counts > 0.0
    ok_ref[...] = jnp.where(written, _DECAY * mk_ref[...] + (1.0 - _DECAY) * means,
                            mk_ref[...])
    ov_ref[...] = jnp.where(written, _DECAY * mv_ref[...] + (1.0 - _DECAY) * means,
                            mv_ref[...])


def _tail(hidden, top_idx, mem_k, mem_v):
    B, S, D = hidden.shape
    m = top_idx.shape[1]
    n_tok = B * m
    flat_rows = (top_idx + jnp.arange(B, dtype=top_idx.dtype)[:, None] * S
                 ).reshape(-1)
    spec = pltpu.PrefetchScalarGridSpec(
        num_scalar_prefetch=1,
        grid=(1,),
        in_specs=[
            pl.BlockSpec((1, n_tok), lambda i, idx: (0, 0)),
            pl.BlockSpec(memory_space=pltpu.MemorySpace.HBM),
            pl.BlockSpec((m, D), lambda i, idx: (0, 0)),
            pl.BlockSpec((m, D), lambda i, idx: (0, 0)),
        ],
        out_specs=[
            pl.BlockSpec((n_tok, D), lambda i, idx: (0, 0)),
            pl.BlockSpec((m, D), lambda i, idx: (0, 0)),
            pl.BlockSpec((m, D), lambda i, idx: (0, 0)),
        ],
        scratch_shapes=[pltpu.SemaphoreType.DMA],
    )
    sk, ok, ov = pl.pallas_call(
        _tail_body,
        grid_spec=spec,
        out_shape=[
            jax.ShapeDtypeStruct((n_tok, D), jnp.float32),
            jax.ShapeDtypeStruct((m, D), jnp.float32),
            jax.ShapeDtypeStruct((m, D), jnp.float32),
        ],
    )(flat_rows, top_idx.reshape(1, n_tok), hidden.reshape(B * S, D),
      mem_k, mem_v)
    return sk.reshape(B, m, D), ok, ov


def kernel(hidden_states, attention_weights, mem_k, mem_v):
    m = mem_k.shape[0]
    scores, ent_mean, ent_var = _fused_scores_entropy(attention_weights)
    _, top_idx = jax.lax.top_k(scores, m)
    sparse_k, new_mem_k, new_mem_v = _tail(hidden_states, top_idx, mem_k, mem_v)
    return (sparse_k, sparse_k, top_idx, new_mem_k, new_mem_v, ent_mean, ent_var)


# speedup vs baseline: 1.0056x; 1.0056x over previous
import jax
import jax.numpy as jnp
from jax.experimental import pallas as pl
from jax.experimental.pallas import tpu as pltpu

_EPS = 1e-9
_DECAY = 0.99

_W = 256


_WPB = 2


def _fused_body(x_ref, sc_ref, mean_ref, var_ref, ent_ref):
    b = pl.program_id(0)
    s = pl.program_id(1)
    nblk = pl.num_programs(1)

    for w in range(_WPB):
        x = x_ref[0, w * _W:(w + 1) * _W, :]

        lacc = x[:, 0:128]
        for c in range(1, x.shape[1] // 128):
            lacc = lacc + x[:, c * 128:(c + 1) * 128]
        s8 = lacc[:, 0:8]
        for k in range(1, 16):
            s8 = s8 + lacc[:, k * 8:(k + 1) * 8]
        a4 = s8[:, 0:4] + s8[:, 4:8]
        a2 = a4[:, 0:2] + a4[:, 2:4]
        tp = (a2[:, 0:1] + a2[:, 1:2]) + _EPS
        attn = x / tp

        acc = attn[0:8, :]
        for k in range(1, _W // 8):
            acc = acc + attn[k * 8:(k + 1) * 8, :]
        a = acc[0:4, :] + acc[4:8, :]
        a = a[0:2, :] + a[2:4, :]
        part = a[0:1, :] + a[1:2, :]

        @pl.when((s == 0) & (w == 0))
        def _():
            sc_ref[0] = part

        if w == 0:
            @pl.when(s > 0)
            def _():
                sc_ref[0] = sc_ref[0] + part
        else:
            sc_ref[0] = sc_ref[0] + part

        e = -jnp.sum(attn * jnp.log(attn + _EPS), axis=-1)
        ent_ref[b * nblk + s, w * _W:(w + 1) * _W] = e

    @pl.when((b == pl.num_programs(0) - 1) & (s == nblk - 1))
    def _():
        ent = ent_ref[...]
        n = ent.size
        mu = jnp.sum(ent) / n
        d = ent - mu
        mean_ref[...] = jnp.reshape(mu, (1, 1))
        var_ref[...] = jnp.reshape(jnp.sum(d * d) / n, (1, 1))


def _fused_scores_entropy(aw):
    B, S, _ = aw.shape
    nblk = S // (_W * _WPB)
    scores, mean, var = pl.pallas_call(
        _fused_body,
        grid=(B, nblk),
        in_specs=[
            pl.BlockSpec((1, _W * _WPB, S), lambda b, s: (b, s, 0)),
        ],
        out_specs=[
            pl.BlockSpec((1, 1, S), lambda b, s: (b, 0, 0)),
            pl.BlockSpec((1, 1), lambda b, s: (0, 0)),
            pl.BlockSpec((1, 1), lambda b, s: (0, 0)),
        ],
        out_shape=[
            jax.ShapeDtypeStruct((B, 1, S), jnp.float32),
            jax.ShapeDtypeStruct((1, 1), jnp.float32),
            jax.ShapeDtypeStruct((1, 1), jnp.float32),
        ],
        scratch_shapes=[pltpu.VMEM((B * nblk, _W * _WPB), jnp.float32)],
    )(aw)
    return scores.reshape(B, S), mean[0, 0], var[0, 0]


def _tail_body(idx_smem, idxv_ref, hid_ref, mk_ref, mv_ref,
               sk_ref, ok_ref, ov_ref, sem):
    n_tok = sk_ref.shape[0]
    m = mk_ref.shape[0]

    def start(i, _):
        row = idx_smem[i]
        pltpu.make_async_copy(hid_ref.at[pl.ds(row, 1), :],
                              sk_ref.at[pl.ds(i, 1), :], sem).start()
        return 0
    jax.lax.fori_loop(0, n_tok, start, 0)

    def wait(i, _):
        pltpu.make_async_copy(hid_ref.at[pl.ds(0, 1), :],
                              sk_ref.at[pl.ds(0, 1), :], sem).wait()
        return 0
    jax.lax.fori_loop(0, n_tok, wait, 0)

    idx = idxv_ref[...]
    slots = jax.lax.rem(idx, m)
    rows = jax.lax.broadcasted_iota(jnp.int32, (m, n_tok), 0)
    oh = (rows == slots).astype(jnp.float32)
    toks = sk_ref[...]
    sums = jnp.dot(oh, toks, preferred_element_type=jnp.float32)
    counts = jnp.sum(oh, axis=1, keepdims=True)
    means = sums / jnp.maximum(counts, 1.0)
    written = counts > 0.0
    ok_ref[...] = jnp.where(written, _DECAY * mk_ref[...] + (1.0 - _DECAY) * means,
                            mk_ref[...])
    ov_ref[...] = jnp.where(written, _DECAY * mv_ref[...] + (1.0 - _DECAY) * means,
                            mv_ref[...])


def _tail(hidden, top_idx, mem_k, mem_v):
    B, S, D = hidden.shape
    m = top_idx.shape[1]
    n_tok = B * m
    flat_rows = (top_idx + jnp.arange(B, dtype=top_idx.dtype)[:, None] * S
                 ).reshape(-1)
    spec = pltpu.PrefetchScalarGridSpec(
        num_scalar_prefetch=1,
        grid=(1,),
        in_specs=[
            pl.BlockSpec((1, n_tok), lambda i, idx: (0, 0)),
            pl.BlockSpec(memory_space=pltpu.MemorySpace.HBM),
            pl.BlockSpec((m, D), lambda i, idx: (0, 0)),
            pl.BlockSpec((m, D), lambda i, idx: (0, 0)),
        ],
        out_specs=[
            pl.BlockSpec((n_tok, D), lambda i, idx: (0, 0)),
            pl.BlockSpec((m, D), lambda i, idx: (0, 0)),
            pl.BlockSpec((m, D), lambda i, idx: (0, 0)),
        ],
        scratch_shapes=[pltpu.SemaphoreType.DMA],
    )
    sk, ok, ov = pl.pallas_call(
        _tail_body,
        grid_spec=spec,
        out_shape=[
            jax.ShapeDtypeStruct((n_tok, D), jnp.float32),
            jax.ShapeDtypeStruct((m, D), jnp.float32),
            jax.ShapeDtypeStruct((m, D), jnp.float32),
        ],
    )(flat_rows, top_idx.reshape(1, n_tok), hidden.reshape(B * S, D),
      mem_k, mem_v)
    return sk.reshape(B, m, D), ok, ov


def kernel(hidden_states, attention_weights, mem_k, mem_v):
    m = mem_k.shape[0]
    scores, ent_mean, ent_var = _fused_scores_entropy(attention_weights)
    _, top_idx = jax.lax.top_k(scores, m)
    sparse_k, new_mem_k, new_mem_v = _tail(hidden_states, top_idx, mem_k, mem_v)
    return (sparse_k, sparse_k, top_idx, new_mem_k, new_mem_v, ent_mean, ent_var)
